# Initial kernel scaffold; baseline (speedup 1.0000x reference)
import jax
import jax.numpy as jnp
from jax.experimental import pallas as pl


def kernel(x, edge_index, W_self, b_self, W_s2d, b_s2d, W_d2s, b_d2s, W_out, b_out):
    raise NotImplementedError("write your pallas kernel here")


# speedup vs baseline: 5.8125x; 5.8125x over previous
import functools

import jax
import jax.numpy as jnp
from jax import lax
from jax.experimental import pallas as pl
from jax.experimental.pallas import tpu as pltpu
from jax.experimental.pallas import tpu_sc as plsc

N = 10000
E = 320000
D = 128
LAYERS = 3
ALPHA = 0.5

NC = 2
NS = 16
CHUNK = 128
NCHUNK = E // CHUNK
ROWS_PER_TILE = 640
ROW_STRIDE = 624
CNT_N = 10240
CNT_PER_TILE = CNT_N // NS


def _sc_agg_body(h_hbm, eidx_hbm, zrows_hbm, out_hbm, idx_v, rows_v, acc_sh, sem):
    c = lax.axis_index("c")
    s = lax.axis_index("s")
    pltpu.sync_copy(zrows_hbm, acc_sh.at[pl.ds(s * ROW_STRIDE, ROWS_PER_TILE)])
    plsc.subcore_barrier()
    lo = (s * NCHUNK) // NS
    hi = ((s + 1) * NCHUNK) // NS

    def body(j, carry):
        pltpu.sync_copy(eidx_hbm.at[:, pl.ds(j * CHUNK, CHUNK)], idx_v)
        pltpu.async_copy(h_hbm.at[idx_v.at[c]], rows_v, sem).wait()
        pltpu.sync_copy(rows_v, acc_sh.at[idx_v.at[1 - c]], add=True)
        return carry

    lax.fori_loop(lo, hi, body, 0)
    plsc.subcore_barrier()
    pltpu.sync_copy(
        acc_sh.at[pl.ds(s * ROW_STRIDE, ROWS_PER_TILE)],
        out_hbm.at[c, pl.ds(s * ROW_STRIDE, ROWS_PER_TILE)],
    )


@jax.jit
def _sc_agg(h, eidx, zrows):
    mesh = plsc.VectorSubcoreMesh(core_axis_name="c", subcore_axis_name="s")
    return pl.kernel(
        _sc_agg_body,
        out_type=jax.ShapeDtypeStruct((NC, N, D), jnp.float32),
        mesh=mesh,
        scratch_types=[
            pltpu.VMEM((NC, CHUNK), jnp.int32),
            pltpu.VMEM((CHUNK, D), jnp.float32),
            pltpu.VMEM_SHARED((N, D), jnp.float32),
            pltpu.SemaphoreType.DMA,
        ],
    )(h, eidx, zrows)


def _sc_cnt_body(eidx_hbm, zcnt_hbm, out_hbm, idx_v, ones_v, cnt_sh):
    c = lax.axis_index("c")
    s = lax.axis_index("s")
    pltpu.sync_copy(zcnt_hbm, cnt_sh.at[pl.ds(s * CNT_PER_TILE, CNT_PER_TILE)])
    for k in range(CHUNK // 16):
        ones_v[pl.ds(k * 16, 16)] = jnp.ones((16,), jnp.float32)
    plsc.subcore_barrier()
    lo = (s * NCHUNK) // NS
    hi = ((s + 1) * NCHUNK) // NS

    def body(j, carry):
        pltpu.sync_copy(eidx_hbm.at[1 - c, pl.ds(j * CHUNK, CHUNK)], idx_v)
        pltpu.sync_copy(ones_v, cnt_sh.at[idx_v], add=True)
        return carry

    lax.fori_loop(lo, hi, body, 0)
    plsc.subcore_barrier()
    pltpu.sync_copy(
        cnt_sh.at[pl.ds(s * CNT_PER_TILE, CNT_PER_TILE)],
        out_hbm.at[c, pl.ds(s * CNT_PER_TILE, CNT_PER_TILE)],
    )


@jax.jit
def _sc_cnt(eidx, zcnt):
    mesh = plsc.VectorSubcoreMesh(core_axis_name="c", subcore_axis_name="s")
    return pl.kernel(
        _sc_cnt_body,
        out_type=jax.ShapeDtypeStruct((NC, CNT_N), jnp.float32),
        mesh=mesh,
        scratch_types=[
            pltpu.VMEM((CHUNK,), jnp.int32),
            pltpu.VMEM((CHUNK,), jnp.float32),
            pltpu.VMEM_SHARED((CNT_N,), jnp.float32),
        ],
    )(eidx, zcnt)


BN = 1000


def _tc_layer_body(h_ref, a0_ref, a1_ref, cd_ref, cs_ref, m_ref,
                   ws_ref, bs_ref, w1_ref, b1_ref, w2_ref, b2_ref,
                   hout_ref, mout_ref):
    inv_d = 1.0 / jnp.maximum(cd_ref[...], 1.0)
    inv_s = 1.0 / jnp.maximum(cs_ref[...], 1.0)
    y = jnp.dot(h_ref[...], ws_ref[...], preferred_element_type=jnp.float32)
    y += bs_ref[...]
    y += (1.0 - ALPHA) * (
        jnp.dot(a0_ref[...] * inv_d, w1_ref[...], preferred_element_type=jnp.float32)
        + b1_ref[...])
    y += ALPHA * (
        jnp.dot(a1_ref[...] * inv_s, w2_ref[...], preferred_element_type=jnp.float32)
        + b2_ref[...])
    h_new = jnp.maximum(y, 0.0)
    hout_ref[...] = h_new
    mout_ref[...] = jnp.maximum(m_ref[...], h_new)


@jax.jit
def _tc_layer(h, a0, a1, cd, cs, m, wst, bs, w1t, b1, w2t, b2):
    f_spec = pl.BlockSpec((BN, D), lambda i: (i, 0))
    w_spec = pl.BlockSpec((D, D), lambda i: (0, 0))
    b_spec = pl.BlockSpec((1, D), lambda i: (0, 0))
    c_spec = pl.BlockSpec((BN, 1), lambda i: (i, 0))
    return pl.pallas_call(
        _tc_layer_body,
        grid=(N // BN,),
        in_specs=[f_spec, f_spec, f_spec, c_spec, c_spec, f_spec,
                  w_spec, b_spec, w_spec, b_spec, w_spec, b_spec],
        out_specs=(f_spec, f_spec),
        out_shape=(jax.ShapeDtypeStruct((N, D), jnp.float32),
                   jax.ShapeDtypeStruct((N, D), jnp.float32)),
    )(h, a0, a1, cd, cs, m, wst, bs, w1t, b1, w2t, b2)


def _tc_final_body(m_ref, w_ref, b_ref, out_ref):
    out_ref[...] = (
        jnp.dot(m_ref[...], w_ref[...], preferred_element_type=jnp.float32)
        + b_ref[...])


@jax.jit
def _tc_final(m, wot, bo):
    f_spec = pl.BlockSpec((BN, D), lambda i: (i, 0))
    w_spec = pl.BlockSpec((D, D), lambda i: (0, 0))
    b_spec = pl.BlockSpec((1, D), lambda i: (0, 0))
    return pl.pallas_call(
        _tc_final_body,
        grid=(N // BN,),
        in_specs=[f_spec, w_spec, b_spec],
        out_specs=f_spec,
        out_shape=jax.ShapeDtypeStruct((N, D), jnp.float32),
    )(m, wot, bo)


def kernel(x, edge_index, W_self, b_self, W_s2d, b_s2d, W_d2s, b_d2s, W_out, b_out):
    zrows = jnp.zeros((ROWS_PER_TILE, D), jnp.float32)
    zcnt = jnp.zeros((CNT_PER_TILE,), jnp.float32)

    cnts = _sc_cnt(edge_index, zcnt)
    cd = cnts[0, :N].reshape(N, 1)
    cs = cnts[1, :N].reshape(N, 1)

    h = x
    m = jnp.zeros((N, D), jnp.float32)
    for i in range(LAYERS):
        agg = _sc_agg(h, edge_index, zrows)
        h, m = _tc_layer(
            h, agg[0], agg[1], cd, cs, m,
            W_self[i].T, b_self[i].reshape(1, D),
            W_s2d[i].T, b_s2d[i].reshape(1, D),
            W_d2s[i].T, b_d2s[i].reshape(1, D),
        )
    return _tc_final(m, W_out.T, b_out.reshape(1, D))


# speedup vs baseline: 9.0687x; 1.5602x over previous
import functools

import jax
import jax.numpy as jnp
from jax import lax
from jax.experimental import pallas as pl
from jax.experimental.pallas import tpu as pltpu
from jax.experimental.pallas import tpu_sc as plsc

N = 10000
E = 320000
D = 128
LAYERS = 3
ALPHA = 0.5

NC = 2
NS = 16
CHUNK = 128
NCHUNK = E // CHUNK
ROWS_PER_TILE = 640
ROW_STRIDE = 624
CNT_N = 10240
CNT_PER_TILE = CNT_N // NS


K = 8
NCHUNK_PAD = 2504
NBLK = NCHUNK_PAD // K


def _sc_agg_body(h_hbm, eidx_hbm, zrows_hbm, out_hbm,
                 ibuf, rows0, rows1, acc_sh, sem0, sem1):
    c = lax.axis_index("c")
    s = lax.axis_index("s")
    pltpu.sync_copy(zrows_hbm, acc_sh.at[pl.ds(s * ROW_STRIDE, ROWS_PER_TILE)])
    plsc.subcore_barrier()
    lo_b = (s * NBLK) // NS
    hi_b = ((s + 1) * NBLK) // NS
    rows = (rows0, rows1)
    sems = (sem0, sem1)

    def blk(jb, carry):
        pltpu.sync_copy(eidx_hbm.at[:, pl.ds(jb * K, K), :], ibuf)
        base = jb * K
        pltpu.async_copy(h_hbm.at[ibuf.at[c, 0]], rows[0], sems[0])
        for k in range(K):
            p = k % 2
            if k + 1 < K:
                @pl.when(base + k + 1 < NCHUNK)
                def _():
                    pltpu.async_copy(h_hbm.at[ibuf.at[c, k + 1]],
                                     rows[1 - p], sems[1 - p])

            @pl.when(base + k < NCHUNK)
            def _():
                pltpu.make_async_copy(h_hbm.at[ibuf.at[c, k]],
                                      rows[p], sems[p]).wait()
                pltpu.sync_copy(rows[p], acc_sh.at[ibuf.at[1 - c, k]], add=True)
        return carry

    lax.fori_loop(lo_b, hi_b, blk, 0)
    plsc.subcore_barrier()
    pltpu.sync_copy(
        acc_sh.at[pl.ds(s * ROW_STRIDE, ROWS_PER_TILE)],
        out_hbm.at[c, pl.ds(s * ROW_STRIDE, ROWS_PER_TILE)],
    )


@jax.jit
def _sc_agg(h, eidx3, zrows):
    mesh = plsc.VectorSubcoreMesh(core_axis_name="c", subcore_axis_name="s")
    return pl.kernel(
        _sc_agg_body,
        out_type=jax.ShapeDtypeStruct((NC, N, D), jnp.float32),
        mesh=mesh,
        scratch_types=[
            pltpu.VMEM((NC, K, CHUNK), jnp.int32),
            pltpu.VMEM((CHUNK, D), jnp.float32),
            pltpu.VMEM((CHUNK, D), jnp.float32),
            pltpu.VMEM_SHARED((N, D), jnp.float32),
            pltpu.SemaphoreType.DMA,
            pltpu.SemaphoreType.DMA,
        ],
    )(h, eidx3, zrows)


def _sc_cnt_body(eidx_hbm, zcnt_hbm, out_hbm, idx_v, ones_v, cnt_sh):
    c = lax.axis_index("c")
    s = lax.axis_index("s")
    pltpu.sync_copy(zcnt_hbm, cnt_sh.at[pl.ds(s * CNT_PER_TILE, CNT_PER_TILE)])
    for k in range(CHUNK // 16):
        ones_v[pl.ds(k * 16, 16)] = jnp.ones((16,), jnp.float32)
    plsc.subcore_barrier()
    lo = (s * NCHUNK) // NS
    hi = ((s + 1) * NCHUNK) // NS

    def body(j, carry):
        pltpu.sync_copy(eidx_hbm.at[1 - c, pl.ds(j * CHUNK, CHUNK)], idx_v)
        pltpu.sync_copy(ones_v, cnt_sh.at[idx_v], add=True)
        return carry

    lax.fori_loop(lo, hi, body, 0)
    plsc.subcore_barrier()
    pltpu.sync_copy(
        cnt_sh.at[pl.ds(s * CNT_PER_TILE, CNT_PER_TILE)],
        out_hbm.at[c, pl.ds(s * CNT_PER_TILE, CNT_PER_TILE)],
    )


@jax.jit
def _sc_cnt(eidx, zcnt):
    mesh = plsc.VectorSubcoreMesh(core_axis_name="c", subcore_axis_name="s")
    return pl.kernel(
        _sc_cnt_body,
        out_type=jax.ShapeDtypeStruct((NC, CNT_N), jnp.float32),
        mesh=mesh,
        scratch_types=[
            pltpu.VMEM((CHUNK,), jnp.int32),
            pltpu.VMEM((CHUNK,), jnp.float32),
            pltpu.VMEM_SHARED((CNT_N,), jnp.float32),
        ],
    )(eidx, zcnt)


BN = 1000


def _tc_layer_body(h_ref, a0_ref, a1_ref, cd_ref, cs_ref, m_ref,
                   ws_ref, bs_ref, w1_ref, b1_ref, w2_ref, b2_ref,
                   hout_ref, mout_ref):
    inv_d = 1.0 / jnp.maximum(cd_ref[...], 1.0)
    inv_s = 1.0 / jnp.maximum(cs_ref[...], 1.0)
    y = jnp.dot(h_ref[...], ws_ref[...], preferred_element_type=jnp.float32)
    y += bs_ref[...]
    y += (1.0 - ALPHA) * (
        jnp.dot(a0_ref[...] * inv_d, w1_ref[...], preferred_element_type=jnp.float32)
        + b1_ref[...])
    y += ALPHA * (
        jnp.dot(a1_ref[...] * inv_s, w2_ref[...], preferred_element_type=jnp.float32)
        + b2_ref[...])
    h_new = jnp.maximum(y, 0.0)
    hout_ref[...] = h_new
    mout_ref[...] = jnp.maximum(m_ref[...], h_new)


@jax.jit
def _tc_layer(h, a0, a1, cd, cs, m, wst, bs, w1t, b1, w2t, b2):
    f_spec = pl.BlockSpec((BN, D), lambda i: (i, 0))
    w_spec = pl.BlockSpec((D, D), lambda i: (0, 0))
    b_spec = pl.BlockSpec((1, D), lambda i: (0, 0))
    c_spec = pl.BlockSpec((BN, 1), lambda i: (i, 0))
    return pl.pallas_call(
        _tc_layer_body,
        grid=(N // BN,),
        in_specs=[f_spec, f_spec, f_spec, c_spec, c_spec, f_spec,
                  w_spec, b_spec, w_spec, b_spec, w_spec, b_spec],
        out_specs=(f_spec, f_spec),
        out_shape=(jax.ShapeDtypeStruct((N, D), jnp.float32),
                   jax.ShapeDtypeStruct((N, D), jnp.float32)),
    )(h, a0, a1, cd, cs, m, wst, bs, w1t, b1, w2t, b2)


def _tc_final_body(m_ref, w_ref, b_ref, out_ref):
    out_ref[...] = (
        jnp.dot(m_ref[...], w_ref[...], preferred_element_type=jnp.float32)
        + b_ref[...])


@jax.jit
def _tc_final(m, wot, bo):
    f_spec = pl.BlockSpec((BN, D), lambda i: (i, 0))
    w_spec = pl.BlockSpec((D, D), lambda i: (0, 0))
    b_spec = pl.BlockSpec((1, D), lambda i: (0, 0))
    return pl.pallas_call(
        _tc_final_body,
        grid=(N // BN,),
        in_specs=[f_spec, w_spec, b_spec],
        out_specs=f_spec,
        out_shape=jax.ShapeDtypeStruct((N, D), jnp.float32),
    )(m, wot, bo)


def kernel(x, edge_index, W_self, b_self, W_s2d, b_s2d, W_d2s, b_d2s, W_out, b_out):
    eidx3 = jnp.pad(edge_index.reshape(2, NCHUNK, CHUNK),
                    ((0, 0), (0, NCHUNK_PAD - NCHUNK), (0, 0)))
    zrows = jnp.zeros((ROWS_PER_TILE, D), jnp.float32)
    zcnt = jnp.zeros((CNT_PER_TILE,), jnp.float32)

    cnts = _sc_cnt(edge_index, zcnt)
    cd = cnts[0, :N].reshape(N, 1)
    cs = cnts[1, :N].reshape(N, 1)

    h = x
    m = jnp.zeros((N, D), jnp.float32)
    for i in range(LAYERS):
        agg = _sc_agg(h, eidx3, zrows)
        h, m = _tc_layer(
            h, agg[0], agg[1], cd, cs, m,
            W_self[i].T, b_self[i].reshape(1, D),
            W_s2d[i].T, b_s2d[i].reshape(1, D),
            W_d2s[i].T, b_d2s[i].reshape(1, D),
        )
    return _tc_final(m, W_out.T, b_out.reshape(1, D))


# speedup vs baseline: 11.5996x; 1.2791x over previous
import functools

import jax
import jax.numpy as jnp
from jax import lax
from jax.experimental import pallas as pl
from jax.experimental.pallas import tpu as pltpu
from jax.experimental.pallas import tpu_sc as plsc

N = 10000
E = 320000
D = 128
LAYERS = 3
ALPHA = 0.5

NC = 2
NS = 16
CHUNK = 128
NCHUNK = E // CHUNK
ROWS_PER_TILE = 640
ROW_STRIDE = 624
CNT_N = 10240
CNT_PER_TILE = CNT_N // NS


K = 8
NCHUNK_PAD = 2504
NBLK = NCHUNK_PAD // K


def _sc_agg_body(h_hbm, eidx_hbm, zrows_hbm, out_hbm,
                 ibuf, rows0, rows1, acc_sh,
                 sem_g0, sem_g1, sem_s0, sem_s1, sem_i):
    c = lax.axis_index("c")
    s = lax.axis_index("s")
    pltpu.sync_copy(zrows_hbm, acc_sh.at[pl.ds(s * ROW_STRIDE, ROWS_PER_TILE)])
    plsc.subcore_barrier()
    lo_b = (s * NBLK) // NS
    hi_b = ((s + 1) * NBLK) // NS
    rows = (rows0, rows1)
    sem_g = (sem_g0, sem_g1)
    sem_s = (sem_s0, sem_s1)

    def _gather(bp, k, p):
        return pltpu.make_async_copy(h_hbm.at[ibuf.at[bp, c, k]], rows[p], sem_g[p])

    def _scatter(bp, k, p):
        return pltpu.make_async_copy(rows[p], acc_sh.at[ibuf.at[bp, 1 - c, k]],
                                     sem_s[p])

    pltpu.sync_copy(eidx_hbm.at[:, pl.ds(lo_b * K, K), :], ibuf.at[0])
    _gather(0, 0, 0).start()

    def blk(jb, carry):
        bp = (jb - lo_b) % 2
        base = jb * K
        for k in range(K):
            p = k % 2
            if k == 0:
                @pl.when(jb > lo_b)
                def _():
                    _scatter(1 - bp, K - 1, 1).wait()

                @pl.when(jb + 1 < hi_b)
                def _():
                    pltpu.async_copy(eidx_hbm.at[:, pl.ds((jb + 1) * K, K), :],
                                     ibuf.at[1 - bp], sem_i)
            else:
                @pl.when(base + k - 1 < NCHUNK)
                def _():
                    _scatter(bp, k - 1, 1 - p).wait()
            if k + 1 < K:
                @pl.when(base + k + 1 < NCHUNK)
                def _():
                    _gather(bp, k + 1, 1 - p).start()
            else:
                @pl.when(jb + 1 < hi_b)
                def _():
                    pltpu.make_async_copy(
                        eidx_hbm.at[:, pl.ds((jb + 1) * K, K), :],
                        ibuf.at[1 - bp], sem_i).wait()
                    _gather(1 - bp, 0, 1 - p).start()

            @pl.when(base + k < NCHUNK)
            def _():
                _gather(bp, k, p).wait()
                _scatter(bp, k, p).start(add=True)
        return carry

    lax.fori_loop(lo_b, hi_b, blk, 0)
    @pl.when(hi_b * K <= NCHUNK)
    def _():
        _scatter((hi_b - 1 - lo_b) % 2, K - 1, 1).wait()

    plsc.subcore_barrier()
    pltpu.sync_copy(
        acc_sh.at[pl.ds(s * ROW_STRIDE, ROWS_PER_TILE)],
        out_hbm.at[c, pl.ds(s * ROW_STRIDE, ROWS_PER_TILE)],
    )


@jax.jit
def _sc_agg(h, eidx3, zrows):
    mesh = plsc.VectorSubcoreMesh(core_axis_name="c", subcore_axis_name="s")
    return pl.kernel(
        _sc_agg_body,
        out_type=jax.ShapeDtypeStruct((NC, N, D), jnp.float32),
        mesh=mesh,
        scratch_types=[
            pltpu.VMEM((2, NC, K, CHUNK), jnp.int32),
            pltpu.VMEM((CHUNK, D), jnp.float32),
            pltpu.VMEM((CHUNK, D), jnp.float32),
            pltpu.VMEM_SHARED((N, D), jnp.float32),
            pltpu.SemaphoreType.DMA,
            pltpu.SemaphoreType.DMA,
            pltpu.SemaphoreType.DMA,
            pltpu.SemaphoreType.DMA,
            pltpu.SemaphoreType.DMA,
        ],
    )(h, eidx3, zrows)


def _sc_cnt_body(eidx_hbm, zcnt_hbm, out_hbm, ibuf, ones_v, cnt_sh, sem):
    c = lax.axis_index("c")
    s = lax.axis_index("s")
    pltpu.sync_copy(zcnt_hbm, cnt_sh.at[pl.ds(s * CNT_PER_TILE, CNT_PER_TILE)])
    for k in range(CHUNK // 16):
        ones_v[pl.ds(k * 16, 16)] = jnp.ones((16,), jnp.float32)
    plsc.subcore_barrier()
    lo_b = (s * NBLK) // NS
    hi_b = ((s + 1) * NBLK) // NS

    def blk(jb, carry):
        pltpu.sync_copy(eidx_hbm.at[1 - c, pl.ds(jb * K, K), :], ibuf)
        base = jb * K
        for k in range(K):
            @pl.when(base + k < NCHUNK)
            def _():
                pltpu.make_async_copy(ones_v, cnt_sh.at[ibuf.at[k]],
                                      sem).start(add=True)
        for k in range(K):
            @pl.when(base + k < NCHUNK)
            def _():
                pltpu.make_async_copy(ones_v, cnt_sh.at[ibuf.at[k]],
                                      sem).wait()
        return carry

    lax.fori_loop(lo_b, hi_b, blk, 0)
    plsc.subcore_barrier()
    pltpu.sync_copy(
        cnt_sh.at[pl.ds(s * CNT_PER_TILE, CNT_PER_TILE)],
        out_hbm.at[c, pl.ds(s * CNT_PER_TILE, CNT_PER_TILE)],
    )


@jax.jit
def _sc_cnt(eidx3, zcnt):
    mesh = plsc.VectorSubcoreMesh(core_axis_name="c", subcore_axis_name="s")
    return pl.kernel(
        _sc_cnt_body,
        out_type=jax.ShapeDtypeStruct((NC, CNT_N), jnp.float32),
        mesh=mesh,
        scratch_types=[
            pltpu.VMEM((K, CHUNK), jnp.int32),
            pltpu.VMEM((CHUNK,), jnp.float32),
            pltpu.VMEM_SHARED((CNT_N,), jnp.float32),
            pltpu.SemaphoreType.DMA,
        ],
    )(eidx3, zcnt)


BN = 1000


def _tc_layer_body(h_ref, a0_ref, a1_ref, cd_ref, cs_ref, m_ref,
                   ws_ref, bs_ref, w1_ref, b1_ref, w2_ref, b2_ref,
                   hout_ref, mout_ref):
    inv_d = 1.0 / jnp.maximum(cd_ref[...], 1.0)
    inv_s = 1.0 / jnp.maximum(cs_ref[...], 1.0)
    y = jnp.dot(h_ref[...], ws_ref[...], preferred_element_type=jnp.float32)
    y += bs_ref[...]
    y += (1.0 - ALPHA) * (
        jnp.dot(a0_ref[0] * inv_d, w1_ref[...], preferred_element_type=jnp.float32)
        + b1_ref[...])
    y += ALPHA * (
        jnp.dot(a1_ref[0] * inv_s, w2_ref[...], preferred_element_type=jnp.float32)
        + b2_ref[...])
    h_new = jnp.maximum(y, 0.0)
    hout_ref[...] = h_new
    mout_ref[...] = jnp.maximum(m_ref[...], h_new)


@jax.jit
def _tc_layer(h, agg, cd, cs, m, wst, bs, w1t, b1, w2t, b2):
    f_spec = pl.BlockSpec((BN, D), lambda i: (i, 0))
    a0_spec = pl.BlockSpec((1, BN, D), lambda i: (0, i, 0))
    a1_spec = pl.BlockSpec((1, BN, D), lambda i: (1, i, 0))
    w_spec = pl.BlockSpec((D, D), lambda i: (0, 0))
    b_spec = pl.BlockSpec((1, D), lambda i: (0, 0))
    c_spec = pl.BlockSpec((BN, 1), lambda i: (i, 0))
    return pl.pallas_call(
        _tc_layer_body,
        grid=(N // BN,),
        in_specs=[f_spec, a0_spec, a1_spec, c_spec, c_spec, f_spec,
                  w_spec, b_spec, w_spec, b_spec, w_spec, b_spec],
        out_specs=(f_spec, f_spec),
        out_shape=(jax.ShapeDtypeStruct((N, D), jnp.float32),
                   jax.ShapeDtypeStruct((N, D), jnp.float32)),
    )(h, agg, agg, cd, cs, m, wst, bs, w1t, b1, w2t, b2)


def _tc_final_body(m_ref, w_ref, b_ref, out_ref):
    out_ref[...] = (
        jnp.dot(m_ref[...], w_ref[...], preferred_element_type=jnp.float32)
        + b_ref[...])


@jax.jit
def _tc_final(m, wot, bo):
    f_spec = pl.BlockSpec((BN, D), lambda i: (i, 0))
    w_spec = pl.BlockSpec((D, D), lambda i: (0, 0))
    b_spec = pl.BlockSpec((1, D), lambda i: (0, 0))
    return pl.pallas_call(
        _tc_final_body,
        grid=(N // BN,),
        in_specs=[f_spec, w_spec, b_spec],
        out_specs=f_spec,
        out_shape=jax.ShapeDtypeStruct((N, D), jnp.float32),
    )(m, wot, bo)


def kernel(x, edge_index, W_self, b_self, W_s2d, b_s2d, W_d2s, b_d2s, W_out, b_out):
    eidx3 = jnp.pad(edge_index.reshape(2, NCHUNK, CHUNK),
                    ((0, 0), (0, NCHUNK_PAD - NCHUNK), (0, 0)))
    zrows = jnp.zeros((ROWS_PER_TILE, D), jnp.float32)
    zcnt = jnp.zeros((CNT_PER_TILE,), jnp.float32)

    cnts = _sc_cnt(eidx3, zcnt)
    cd = cnts[0, :N].reshape(N, 1)
    cs = cnts[1, :N].reshape(N, 1)

    h = x
    m = jnp.zeros((N, D), jnp.float32)
    for i in range(LAYERS):
        agg = _sc_agg(h, eidx3, zrows)
        h, m = _tc_layer(
            h, agg, cd, cs, m,
            W_self[i].T, b_self[i].reshape(1, D),
            W_s2d[i].T, b_s2d[i].reshape(1, D),
            W_d2s[i].T, b_d2s[i].reshape(1, D),
        )
    return _tc_final(m, W_out.T, b_out.reshape(1, D))


# speedup vs baseline: 12.0763x; 1.0411x over previous
import functools

import jax
import jax.numpy as jnp
from jax import lax
from jax.experimental import pallas as pl
from jax.experimental.pallas import tpu as pltpu
from jax.experimental.pallas import tpu_sc as plsc

N = 10000
E = 320000
D = 128
ALPHA = 0.5

NC = 2
NS = 16
CHUNK = 128
NCHUNK = E // CHUNK
K = 8
NCHUNK_PAD = 2504
NBLK = NCHUNK_PAD // K
ROWS_PER_TILE = 640
ROW_STRIDE = 624
CNT_N = 10240
CNT_PER_TILE = CNT_N // NS


def _sc_agg_body(with_counts, *refs):
    if with_counts:
        (h_hbm, eidx_hbm, zrows_hbm, zcnt_hbm, out_hbm, cnt_hbm,
         ibuf, rows0, rows1, acc_sh,
         sem_g0, sem_g1, sem_s0, sem_s1, sem_i,
         ones_v, cnt_sh, sem_c) = refs
    else:
        (h_hbm, eidx_hbm, zrows_hbm, out_hbm,
         ibuf, rows0, rows1, acc_sh,
         sem_g0, sem_g1, sem_s0, sem_s1, sem_i) = refs
    c = lax.axis_index("c")
    s = lax.axis_index("s")
    pltpu.sync_copy(zrows_hbm, acc_sh.at[pl.ds(s * ROW_STRIDE, ROWS_PER_TILE)])
    if with_counts:
        pltpu.sync_copy(zcnt_hbm, cnt_sh.at[pl.ds(s * CNT_PER_TILE, CNT_PER_TILE)])
        for k in range(CHUNK // 16):
            ones_v[pl.ds(k * 16, 16)] = jnp.ones((16,), jnp.float32)
    plsc.subcore_barrier()
    lo_b = (s * NBLK) // NS
    hi_b = ((s + 1) * NBLK) // NS
    rows = (rows0, rows1)
    sem_g = (sem_g0, sem_g1)
    sem_s = (sem_s0, sem_s1)

    def _gather(bp, k, p):
        return pltpu.make_async_copy(h_hbm.at[ibuf.at[bp, c, k]], rows[p], sem_g[p])

    def _scatter(bp, k, p):
        return pltpu.make_async_copy(rows[p], acc_sh.at[ibuf.at[bp, 1 - c, k]],
                                     sem_s[p])

    def _cscatter(bp, k):
        return pltpu.make_async_copy(ones_v, cnt_sh.at[ibuf.at[bp, 1 - c, k]],
                                     sem_c)

    pltpu.sync_copy(eidx_hbm.at[:, pl.ds(lo_b * K, K), :], ibuf.at[0])
    _gather(0, 0, 0).start()

    def blk(jb, carry):
        bp = (jb - lo_b) % 2
        base = jb * K
        for k in range(K):
            p = k % 2
            if k == 0:
                @pl.when(jb > lo_b)
                def _():
                    _scatter(1 - bp, K - 1, 1).wait()
                    if with_counts:
                        for kk in range(K):
                            _cscatter(1 - bp, kk).wait()

                @pl.when(jb + 1 < hi_b)
                def _():
                    pltpu.async_copy(eidx_hbm.at[:, pl.ds((jb + 1) * K, K), :],
                                     ibuf.at[1 - bp], sem_i)
            else:
                @pl.when(base + k - 1 < NCHUNK)
                def _():
                    _scatter(bp, k - 1, 1 - p).wait()
            if k + 1 < K:
                @pl.when(base + k + 1 < NCHUNK)
                def _():
                    _gather(bp, k + 1, 1 - p).start()
            else:
                @pl.when(jb + 1 < hi_b)
                def _():
                    pltpu.make_async_copy(
                        eidx_hbm.at[:, pl.ds((jb + 1) * K, K), :],
                        ibuf.at[1 - bp], sem_i).wait()
                    _gather(1 - bp, 0, 1 - p).start()

            @pl.when(base + k < NCHUNK)
            def _():
                _gather(bp, k, p).wait()
                _scatter(bp, k, p).start(add=True)
                if with_counts:
                    _cscatter(bp, k).start(add=True)
        return carry

    lax.fori_loop(lo_b, hi_b, blk, 0)
    @pl.when(hi_b * K <= NCHUNK)
    def _():
        _scatter((hi_b - 1 - lo_b) % 2, K - 1, 1).wait()

    if with_counts:
        last_bp = (hi_b - 1 - lo_b) % 2
        for kk in range(K):
            @pl.when((hi_b - 1) * K + kk < NCHUNK)
            def _():
                _cscatter(last_bp, kk).wait()

    plsc.subcore_barrier()
    pltpu.sync_copy(
        acc_sh.at[pl.ds(s * ROW_STRIDE, ROWS_PER_TILE)],
        out_hbm.at[c, pl.ds(s * ROW_STRIDE, ROWS_PER_TILE)],
    )
    if with_counts:
        pltpu.sync_copy(
            cnt_sh.at[pl.ds(s * CNT_PER_TILE, CNT_PER_TILE)],
            cnt_hbm.at[c, pl.ds(s * CNT_PER_TILE, CNT_PER_TILE)],
        )


_AGG_SCRATCH = [
    pltpu.VMEM((2, NC, K, CHUNK), jnp.int32),
    pltpu.VMEM((CHUNK, D), jnp.float32),
    pltpu.VMEM((CHUNK, D), jnp.float32),
    pltpu.VMEM_SHARED((N, D), jnp.float32),
    pltpu.SemaphoreType.DMA,
    pltpu.SemaphoreType.DMA,
    pltpu.SemaphoreType.DMA,
    pltpu.SemaphoreType.DMA,
    pltpu.SemaphoreType.DMA,
]


@jax.jit
def _sc_agg(h, eidx3, zrows):
    mesh = plsc.VectorSubcoreMesh(core_axis_name="c", subcore_axis_name="s")
    return pl.kernel(
        functools.partial(_sc_agg_body, False),
        out_type=jax.ShapeDtypeStruct((NC, N, D), jnp.float32),
        mesh=mesh,
        scratch_types=list(_AGG_SCRATCH),
    )(h, eidx3, zrows)


@jax.jit
def _sc_agg_cnt(h, eidx3, zrows, zcnt):
    mesh = plsc.VectorSubcoreMesh(core_axis_name="c", subcore_axis_name="s")
    return pl.kernel(
        functools.partial(_sc_agg_body, True),
        out_type=(jax.ShapeDtypeStruct((NC, N, D), jnp.float32),
                  jax.ShapeDtypeStruct((NC, CNT_N), jnp.float32)),
        mesh=mesh,
        scratch_types=list(_AGG_SCRATCH) + [
            pltpu.VMEM((CHUNK,), jnp.float32),
            pltpu.VMEM_SHARED((CNT_N,), jnp.float32),
            pltpu.SemaphoreType.DMA,
        ],
    )(h, eidx3, zrows, zcnt)


BN = 1000


def _tc_layer_body(has_m, final, *refs):
    if final:
        (h_ref, a0_ref, a1_ref, cd_ref, cs_ref, m_ref,
         ws_ref, bs_ref, w1_ref, b1_ref, w2_ref, b2_ref,
         wo_ref, bo_ref, out_ref) = refs
    elif has_m:
        (h_ref, a0_ref, a1_ref, cd_ref, cs_ref, m_ref,
         ws_ref, bs_ref, w1_ref, b1_ref, w2_ref, b2_ref,
         hout_ref, mout_ref) = refs
    else:
        (h_ref, a0_ref, a1_ref, cd_ref, cs_ref,
         ws_ref, bs_ref, w1_ref, b1_ref, w2_ref, b2_ref,
         hout_ref) = refs
    inv_d = 1.0 / jnp.maximum(cd_ref[...], 1.0)
    inv_s = 1.0 / jnp.maximum(cs_ref[...], 1.0)
    y = jnp.dot(h_ref[...], ws_ref[...], preferred_element_type=jnp.float32)
    y += bs_ref[...]
    y += (1.0 - ALPHA) * (
        jnp.dot(a0_ref[0] * inv_d, w1_ref[...], preferred_element_type=jnp.float32)
        + b1_ref[...])
    y += ALPHA * (
        jnp.dot(a1_ref[0] * inv_s, w2_ref[...], preferred_element_type=jnp.float32)
        + b2_ref[...])
    h_new = jnp.maximum(y, 0.0)
    if final:
        m_new = jnp.maximum(m_ref[...], h_new)
        out_ref[...] = (
            jnp.dot(m_new, wo_ref[...], preferred_element_type=jnp.float32)
            + bo_ref[...])
    elif has_m:
        hout_ref[...] = h_new
        mout_ref[...] = jnp.maximum(m_ref[...], h_new)
    else:
        hout_ref[...] = h_new


_F_SPEC = pl.BlockSpec((BN, D), lambda i: (i, 0))
_A0_SPEC = pl.BlockSpec((1, BN, D), lambda i: (0, i, 0))
_A1_SPEC = pl.BlockSpec((1, BN, D), lambda i: (1, i, 0))
_W_SPEC = pl.BlockSpec((D, D), lambda i: (0, 0))
_B_SPEC = pl.BlockSpec((1, D), lambda i: (0, 0))
_C_SPEC = pl.BlockSpec((BN, 1), lambda i: (i, 0))
_FOUT = jax.ShapeDtypeStruct((N, D), jnp.float32)


@jax.jit
def _tc_layer_first(h, agg, cd, cs, wst, bs, w1t, b1, w2t, b2):
    return pl.pallas_call(
        functools.partial(_tc_layer_body, False, False),
        grid=(N // BN,),
        in_specs=[_F_SPEC, _A0_SPEC, _A1_SPEC, _C_SPEC, _C_SPEC,
                  _W_SPEC, _B_SPEC, _W_SPEC, _B_SPEC, _W_SPEC, _B_SPEC],
        out_specs=_F_SPEC,
        out_shape=_FOUT,
    )(h, agg, agg, cd, cs, wst, bs, w1t, b1, w2t, b2)


@jax.jit
def _tc_layer_mid(h, agg, cd, cs, m, wst, bs, w1t, b1, w2t, b2):
    return pl.pallas_call(
        functools.partial(_tc_layer_body, True, False),
        grid=(N // BN,),
        in_specs=[_F_SPEC, _A0_SPEC, _A1_SPEC, _C_SPEC, _C_SPEC, _F_SPEC,
                  _W_SPEC, _B_SPEC, _W_SPEC, _B_SPEC, _W_SPEC, _B_SPEC],
        out_specs=(_F_SPEC, _F_SPEC),
        out_shape=(_FOUT, _FOUT),
    )(h, agg, agg, cd, cs, m, wst, bs, w1t, b1, w2t, b2)


@jax.jit
def _tc_layer_last(h, agg, cd, cs, m, wst, bs, w1t, b1, w2t, b2, wot, bo):
    return pl.pallas_call(
        functools.partial(_tc_layer_body, True, True),
        grid=(N // BN,),
        in_specs=[_F_SPEC, _A0_SPEC, _A1_SPEC, _C_SPEC, _C_SPEC, _F_SPEC,
                  _W_SPEC, _B_SPEC, _W_SPEC, _B_SPEC, _W_SPEC, _B_SPEC,
                  _W_SPEC, _B_SPEC],
        out_specs=_F_SPEC,
        out_shape=_FOUT,
    )(h, agg, agg, cd, cs, m, wst, bs, w1t, b1, w2t, b2, wot, bo)


def kernel(x, edge_index, W_self, b_self, W_s2d, b_s2d, W_d2s, b_d2s, W_out, b_out):
    eidx3 = jnp.pad(edge_index.reshape(2, NCHUNK, CHUNK),
                    ((0, 0), (0, NCHUNK_PAD - NCHUNK), (0, 0)))
    zrows = jnp.zeros((ROWS_PER_TILE, D), jnp.float32)
    zcnt = jnp.zeros((CNT_PER_TILE,), jnp.float32)

    def wb(i, W, b):
        return W[i].T, b[i].reshape(1, D)

    agg, cnts = _sc_agg_cnt(x, eidx3, zrows, zcnt)
    cd = cnts[0, :N].reshape(N, 1)
    cs = cnts[1, :N].reshape(N, 1)

    h1 = _tc_layer_first(x, agg, cd, cs,
                         *wb(0, W_self, b_self), *wb(0, W_s2d, b_s2d),
                         *wb(0, W_d2s, b_d2s))
    agg = _sc_agg(h1, eidx3, zrows)
    h2, m2 = _tc_layer_mid(h1, agg, cd, cs, h1,
                           *wb(1, W_self, b_self), *wb(1, W_s2d, b_s2d),
                           *wb(1, W_d2s, b_d2s))
    agg = _sc_agg(h2, eidx3, zrows)
    return _tc_layer_last(h2, agg, cd, cs, m2,
                          *wb(2, W_self, b_self), *wb(2, W_s2d, b_s2d),
                          *wb(2, W_d2s, b_d2s), W_out.T, b_out.reshape(1, D))
